# baseline (speedup 1.0000x reference)
import functools

import jax
import jax.numpy as jnp
from jax import lax
from jax.experimental import pallas as pl
from jax.experimental.pallas import tpu as pltpu
from jax.experimental.pallas import tpu_sc as plsc

N = 10000
NP = 10240
D = 128
E = 320000
NC = 2
NS = 16
NW = NC * NS
EPW = E // NW
CS = 64
WCH = 32
NWIN = 5
EPAD = NWIN * WCH * CS - EPW
RPS = NP // NS
BN = 1024

_mesh = plsc.VectorSubcoreMesh(core_axis_name="c", subcore_axis_name="s")


def _make_agg(compute_deg):
  out_type = [jax.ShapeDtypeStruct((NC, NP, D), jnp.float32)]
  scratch = [
      pltpu.VMEM((WCH, CS), jnp.int32),
      pltpu.VMEM((WCH, CS), jnp.int32),
      pltpu.VMEM((2, CS, D), jnp.float32),
      pltpu.VMEM_SHARED((NP, D), jnp.float32),
      pltpu.SemaphoreType.DMA,
      pltpu.SemaphoreType.DMA,
  ]
  if compute_deg:
    out_type.append(jax.ShapeDtypeStruct((NW, NP), jnp.float32))
    scratch.insert(3, pltpu.VMEM((NP,), jnp.float32))

  @functools.partial(pl.kernel, mesh=_mesh, out_type=out_type,
                     scratch_types=scratch)
  def agg(*refs):
    if compute_deg:
      (x_hbm, src_hbm, dst_hbm, aggr_out, deg_out,
       src_v, dst_v, rows_v, deg_v, acc_sh, sem_g, sem_s) = refs
    else:
      (x_hbm, src_hbm, dst_hbm, aggr_out,
       src_v, dst_v, rows_v, acc_sh, sem_g, sem_s) = refs

    c = lax.axis_index("c")
    s = lax.axis_index("s")
    w = c * NS + s

    z16 = jnp.zeros((16,), jnp.float32)

    def zrow(i, carry):
      for l in range(D // 16):
        rows_v[0, i, pl.ds(l * 16, 16)] = z16
      return carry
    lax.fori_loop(0, CS, zrow, 0)
    for t in range(RPS // CS):
      pltpu.sync_copy(rows_v.at[0], acc_sh.at[pl.ds(s * RPS + t * CS, CS)])
    if compute_deg:
      def zdeg(i, carry):
        deg_v[pl.ds(i * 16, 16)] = z16
        return carry
      lax.fori_loop(0, NP // 16, zdeg, 0)
    plsc.subcore_barrier()

    onehot = jnp.where(lax.iota(jnp.int32, 16) == 0, 1.0, 0.0)

    def hist(j):
      if compute_deg:
        for l in range(CS // 16):
          dvec = dst_v[j, pl.ds(l * 16, 16)]
          for q in range(16):
            d = dvec[q]
            deg_v[pl.ds(d, 16)] = deg_v[pl.ds(d, 16)] + onehot

    def window(t, carry):
      pltpu.sync_copy(src_hbm.at[w, t], src_v)
      pltpu.sync_copy(dst_hbm.at[w, t], dst_v)
      pltpu.async_copy(x_hbm.at[src_v.at[0]], rows_v.at[0], sem_g).wait()

      def chunk(j, carry2):
        p = lax.rem(j, 2)
        cpg = pltpu.async_copy(x_hbm.at[src_v.at[j + 1]], rows_v.at[1 - p],
                               sem_g)
        cps = pltpu.async_copy(rows_v.at[p], acc_sh.at[dst_v.at[j]], sem_s,
                               add=True)
        hist(j)
        cps.wait()
        cpg.wait()
        return carry2
      lax.fori_loop(0, WCH - 1, chunk, 0)
      pltpu.async_copy(rows_v.at[(WCH - 1) % 2], acc_sh.at[dst_v.at[WCH - 1]],
                       sem_s, add=True).wait()
      hist(WCH - 1)
      return carry
    lax.fori_loop(0, NWIN, window, 0)

    plsc.subcore_barrier()
    pltpu.sync_copy(acc_sh.at[pl.ds(s * RPS, RPS)],
                    aggr_out.at[c, pl.ds(s * RPS, RPS)])
    if compute_deg:
      pltpu.sync_copy(deg_v, deg_out.at[w])

  return agg


_agg_with_deg = _make_agg(True)
_agg_no_deg = _make_agg(False)


def _mm_body(relu, aggrp_ref, deg_ref, x_ref, wl_ref, wr_ref, b_ref, out_ref):
  a = aggrp_ref[0] + aggrp_ref[1]
  deg = jnp.sum(deg_ref[...], axis=0)
  inv = 1.0 / jnp.maximum(deg, 1.0)
  m = a * inv[:, None]
  acc = jnp.dot(m, wl_ref[...], preferred_element_type=jnp.float32)
  acc = acc + jnp.dot(x_ref[...], wr_ref[...],
                      preferred_element_type=jnp.float32)
  acc = acc + b_ref[...]
  if relu:
    acc = jnp.maximum(acc, 0.0)
  out_ref[...] = acc


def _mm(aggrp, degp, x, wl, wr, b, relu):
  grid = (pl.cdiv(N, BN),)
  return pl.pallas_call(
      functools.partial(_mm_body, relu),
      grid=grid,
      in_specs=[
          pl.BlockSpec((NC, BN, D), lambda i: (0, i, 0)),
          pl.BlockSpec((NW, BN), lambda i: (0, i)),
          pl.BlockSpec((BN, D), lambda i: (i, 0)),
          pl.BlockSpec((D, D), lambda i: (0, 0)),
          pl.BlockSpec((D, D), lambda i: (0, 0)),
          pl.BlockSpec((1, D), lambda i: (0, 0)),
      ],
      out_specs=pl.BlockSpec((BN, D), lambda i: (i, 0)),
      out_shape=jax.ShapeDtypeStruct((N, D), jnp.float32),
  )(aggrp, degp, x, wl, wr, b)


def kernel(x, edge_index, layer, Wl_stack, Wr_stack, b_stack, W2_l, W2_r, b2):
  src = edge_index[0].astype(jnp.int32).reshape(NW, EPW)
  dst = edge_index[1].astype(jnp.int32).reshape(NW, EPW)
  src = jnp.concatenate(
      [src, jnp.zeros((NW, EPAD), jnp.int32)], axis=1
  ).reshape(NW, NWIN, WCH, CS)
  dst = jnp.concatenate(
      [dst, jnp.full((NW, EPAD), N, jnp.int32)], axis=1
  ).reshape(NW, NWIN, WCH, CS)
  Wl = Wl_stack[layer]
  Wr = Wr_stack[layer]
  b = b_stack[layer]

  aggr1, degp = _agg_with_deg(x, src, dst)
  h = _mm(aggr1, degp, x, Wl, Wr, b.reshape(1, D), relu=True)
  (aggr2,) = _agg_no_deg(h, src, dst)
  out = _mm(aggr2, degp, h, W2_l, W2_r, b2.reshape(1, D), relu=False)
  return out


# speedup vs baseline: 1.0892x; 1.0892x over previous
import functools

import jax
import jax.numpy as jnp
from jax import lax
from jax.experimental import pallas as pl
from jax.experimental.pallas import tpu as pltpu
from jax.experimental.pallas import tpu_sc as plsc

N = 10000
NP = 10240
D = 128
E = 320000
NC = 2
NS = 16
NW = NC * NS
EPW = E // NW
CS = 128
WCH = 16
NWIN = 5
EPAD = 240
RPS = NP // NS
BN = 1024

_mesh = plsc.VectorSubcoreMesh(core_axis_name="c", subcore_axis_name="s")


def _make_agg(compute_deg):
  out_type = [jax.ShapeDtypeStruct((NC, NP, D), jnp.float32)]
  scratch = [
      pltpu.VMEM((WCH, CS), jnp.int32),
      pltpu.VMEM((WCH, CS), jnp.int32),
      pltpu.VMEM((2, CS, D), jnp.float32),
      pltpu.VMEM_SHARED((NP, D), jnp.float32),
      pltpu.SemaphoreType.DMA,
      pltpu.SemaphoreType.DMA,
  ]
  if compute_deg:
    out_type.append(jax.ShapeDtypeStruct((NW, NP), jnp.float32))
    scratch.insert(3, pltpu.VMEM((NP,), jnp.float32))

  @functools.partial(pl.kernel, mesh=_mesh, out_type=out_type,
                     scratch_types=scratch)
  def agg(*refs):
    if compute_deg:
      (x_hbm, src_hbm, dst_hbm, aggr_out, deg_out,
       src_v, dst_v, rows_v, deg_v, acc_sh, sem_g, sem_s) = refs
    else:
      (x_hbm, src_hbm, dst_hbm, aggr_out,
       src_v, dst_v, rows_v, acc_sh, sem_g, sem_s) = refs

    c = lax.axis_index("c")
    s = lax.axis_index("s")
    w = c * NS + s

    z16 = jnp.zeros((16,), jnp.float32)

    def zrow(i, carry):
      for l in range(D // 16):
        rows_v[0, i, pl.ds(l * 16, 16)] = z16
      return carry
    lax.fori_loop(0, CS, zrow, 0)
    for t in range(RPS // CS):
      pltpu.sync_copy(rows_v.at[0], acc_sh.at[pl.ds(s * RPS + t * CS, CS)])
    if compute_deg:
      def zdeg(i, carry):
        deg_v[pl.ds(i * 16, 16)] = z16
        return carry
      lax.fori_loop(0, NP // 16, zdeg, 0)
    plsc.subcore_barrier()

    onehot = jnp.where(lax.iota(jnp.int32, 16) == 0, 1.0, 0.0)

    def hist(j):
      if compute_deg:
        for l in range(CS // 16):
          dvec = dst_v[j, pl.ds(l * 16, 16)]
          for q in range(16):
            d = dvec[q]
            deg_v[pl.ds(d, 16)] = deg_v[pl.ds(d, 16)] + onehot

    def window(t, carry):
      pltpu.sync_copy(src_hbm.at[w, t], src_v)
      pltpu.sync_copy(dst_hbm.at[w, t], dst_v)
      pltpu.async_copy(x_hbm.at[src_v.at[0]], rows_v.at[0], sem_g).wait()

      def chunk(j, carry2):
        p = lax.rem(j, 2)
        cpg = pltpu.async_copy(x_hbm.at[src_v.at[j + 1]], rows_v.at[1 - p],
                               sem_g)
        cps = pltpu.async_copy(rows_v.at[p], acc_sh.at[dst_v.at[j]], sem_s,
                               add=True)
        hist(j)
        cps.wait()
        cpg.wait()
        return carry2
      lax.fori_loop(0, WCH - 1, chunk, 0)
      pltpu.async_copy(rows_v.at[(WCH - 1) % 2], acc_sh.at[dst_v.at[WCH - 1]],
                       sem_s, add=True).wait()
      hist(WCH - 1)
      return carry
    lax.fori_loop(0, NWIN, window, 0)

    plsc.subcore_barrier()
    pltpu.sync_copy(acc_sh.at[pl.ds(s * RPS, RPS)],
                    aggr_out.at[c, pl.ds(s * RPS, RPS)])
    if compute_deg:
      pltpu.sync_copy(deg_v, deg_out.at[w])

  return agg


_agg_with_deg = _make_agg(True)
_agg_no_deg = _make_agg(False)


def _mm_body(relu, aggrp_ref, deg_ref, x_ref, wl_ref, wr_ref, b_ref, out_ref):
  a = aggrp_ref[0] + aggrp_ref[1]
  deg = jnp.sum(deg_ref[...], axis=0)
  inv = 1.0 / jnp.maximum(deg, 1.0)
  m = a * inv[:, None]
  acc = jnp.dot(m, wl_ref[...], preferred_element_type=jnp.float32)
  acc = acc + jnp.dot(x_ref[...], wr_ref[...],
                      preferred_element_type=jnp.float32)
  acc = acc + b_ref[...]
  if relu:
    acc = jnp.maximum(acc, 0.0)
  out_ref[...] = acc


def _mm(aggrp, degp, x, wl, wr, b, relu):
  grid = (pl.cdiv(N, BN),)
  return pl.pallas_call(
      functools.partial(_mm_body, relu),
      grid=grid,
      in_specs=[
          pl.BlockSpec((NC, BN, D), lambda i: (0, i, 0)),
          pl.BlockSpec((NW, BN), lambda i: (0, i)),
          pl.BlockSpec((BN, D), lambda i: (i, 0)),
          pl.BlockSpec((D, D), lambda i: (0, 0)),
          pl.BlockSpec((D, D), lambda i: (0, 0)),
          pl.BlockSpec((1, D), lambda i: (0, 0)),
      ],
      out_specs=pl.BlockSpec((BN, D), lambda i: (i, 0)),
      out_shape=jax.ShapeDtypeStruct((N, D), jnp.float32),
  )(aggrp, degp, x, wl, wr, b)


def kernel(x, edge_index, layer, Wl_stack, Wr_stack, b_stack, W2_l, W2_r, b2):
  src = edge_index[0].astype(jnp.int32).reshape(NW, EPW)
  dst = edge_index[1].astype(jnp.int32).reshape(NW, EPW)
  pad_dst = N + (jnp.arange(EPAD, dtype=jnp.int32) % 224)
  src = jnp.concatenate(
      [src, jnp.zeros((NW, EPAD), jnp.int32)], axis=1
  ).reshape(NW, NWIN, WCH, CS)
  dst = jnp.concatenate(
      [dst, jnp.broadcast_to(pad_dst, (NW, EPAD))], axis=1
  ).reshape(NW, NWIN, WCH, CS)
  Wl = Wl_stack[layer]
  Wr = Wr_stack[layer]
  b = b_stack[layer]

  aggr1, degp = _agg_with_deg(x, src, dst)
  h = _mm(aggr1, degp, x, Wl, Wr, b.reshape(1, D), relu=True)
  (aggr2,) = _agg_no_deg(h, src, dst)
  out = _mm(aggr2, degp, h, W2_l, W2_r, b2.reshape(1, D), relu=False)
  return out


# speedup vs baseline: 2.9047x; 2.6668x over previous
import functools

import jax
import jax.numpy as jnp
from jax import lax
from jax.experimental import pallas as pl
from jax.experimental.pallas import tpu as pltpu
from jax.experimental.pallas import tpu_sc as plsc

N = 10000
NP = 10240
D = 128
E = 320000
NC = 2
NS = 16
NW = NC * NS
EPW = E // NW
CS = 128
WCH = 16
NWIN = 5
EPAD = 240
RPS = NP // NS
BN = 1024

_mesh = plsc.VectorSubcoreMesh(core_axis_name="c", subcore_axis_name="s")


def _make_agg(compute_deg):
  out_type = [jax.ShapeDtypeStruct((NC, NP, D), jnp.float32)]
  scratch = [
      pltpu.VMEM((WCH, CS), jnp.int32),
      pltpu.VMEM((WCH, CS), jnp.int32),
      pltpu.VMEM((2, CS, D), jnp.float32),
      pltpu.VMEM_SHARED((NP, D), jnp.float32),
      pltpu.SemaphoreType.DMA,
      pltpu.SemaphoreType.DMA,
  ]
  if compute_deg:
    out_type.append(jax.ShapeDtypeStruct((NW, NP), jnp.float32))
    scratch.insert(3, pltpu.VMEM((NP,), jnp.float32))

  @functools.partial(pl.kernel, mesh=_mesh, out_type=out_type,
                     scratch_types=scratch)
  def agg(*refs):
    if compute_deg:
      (x_hbm, src_hbm, dst_hbm, aggr_out, deg_out,
       src_v, dst_v, rows_v, deg_v, acc_sh, sem_g, sem_s) = refs
    else:
      (x_hbm, src_hbm, dst_hbm, aggr_out,
       src_v, dst_v, rows_v, acc_sh, sem_g, sem_s) = refs

    c = lax.axis_index("c")
    s = lax.axis_index("s")
    w = c * NS + s

    z16 = jnp.zeros((16,), jnp.float32)

    def zrow(i, carry):
      for l in range(D // 16):
        rows_v[0, i, pl.ds(l * 16, 16)] = z16
      return carry
    lax.fori_loop(0, CS, zrow, 0)
    for t in range(RPS // CS):
      pltpu.sync_copy(rows_v.at[0], acc_sh.at[pl.ds(s * RPS + t * CS, CS)])
    if compute_deg:
      def zdeg(i, carry):
        deg_v[pl.ds(i * 16, 16)] = z16
        return carry
      lax.fori_loop(0, NP // 16, zdeg, 0)
    plsc.subcore_barrier()

    onehot = jnp.where(lax.iota(jnp.int32, 16) == 0, 1.0, 0.0)

    def hist(j):
      if compute_deg:
        for l in range(CS // 16):
          dvec = dst_v[j, pl.ds(l * 16, 16)]
          for q in range(16):
            d = dvec[q]
            deg_v[pl.ds(d, 16)] = deg_v[pl.ds(d, 16)] + onehot

    def window(t, carry):
      pltpu.sync_copy(src_hbm.at[w, t], src_v)
      pltpu.sync_copy(dst_hbm.at[w, t], dst_v)
      pltpu.async_copy(x_hbm.at[src_v.at[0]], rows_v.at[0], sem_g).wait()

      def chunk(j, carry2):
        p = lax.rem(j, 2)
        cpg = pltpu.async_copy(x_hbm.at[src_v.at[j + 1]], rows_v.at[1 - p],
                               sem_g)
        cps = pltpu.async_copy(rows_v.at[p], acc_sh.at[dst_v.at[j]], sem_s,
                               add=True)
        hist(j)
        cps.wait()
        cpg.wait()
        return carry2
      lax.fori_loop(0, WCH - 1, chunk, 0)
      pltpu.async_copy(rows_v.at[(WCH - 1) % 2], acc_sh.at[dst_v.at[WCH - 1]],
                       sem_s, add=True).wait()
      hist(WCH - 1)
      return carry
    lax.fori_loop(0, NWIN, window, 0)

    plsc.subcore_barrier()
    pltpu.sync_copy(acc_sh.at[pl.ds(s * RPS, RPS)],
                    aggr_out.at[c, pl.ds(s * RPS, RPS)])
    if compute_deg:
      pltpu.sync_copy(deg_v, deg_out.at[w])

  return agg


_agg_with_deg = _make_agg(True)
_agg_no_deg = _make_agg(False)


def _mm_body(relu, aggrp_ref, deg_ref, x_ref, wl_ref, wr_ref, b_ref, out_ref):
  a = aggrp_ref[0] + aggrp_ref[1]
  deg = jnp.sum(deg_ref[...], axis=0)
  inv = 1.0 / jnp.maximum(deg, 1.0)
  m = a * inv[:, None]
  acc = jnp.dot(m, wl_ref[...], preferred_element_type=jnp.float32)
  acc = acc + jnp.dot(x_ref[...], wr_ref[...],
                      preferred_element_type=jnp.float32)
  acc = acc + b_ref[...]
  if relu:
    acc = jnp.maximum(acc, 0.0)
  out_ref[...] = acc


def _mm(aggrp, degp, x, wl, wr, b, relu):
  grid = (pl.cdiv(N, BN),)
  return pl.pallas_call(
      functools.partial(_mm_body, relu),
      grid=grid,
      in_specs=[
          pl.BlockSpec((NC, BN, D), lambda i: (0, i, 0)),
          pl.BlockSpec((NW, BN), lambda i: (0, i)),
          pl.BlockSpec((BN, D), lambda i: (i, 0)),
          pl.BlockSpec((D, D), lambda i: (0, 0)),
          pl.BlockSpec((D, D), lambda i: (0, 0)),
          pl.BlockSpec((1, D), lambda i: (0, 0)),
      ],
      out_specs=pl.BlockSpec((BN, D), lambda i: (i, 0)),
      out_shape=jax.ShapeDtypeStruct((N, D), jnp.float32),
  )(aggrp, degp, x, wl, wr, b)


def kernel(x, edge_index, layer, Wl_stack, Wr_stack, b_stack, W2_l, W2_r, b2):
  src = edge_index[0].astype(jnp.int32).reshape(NW, EPW)
  dst = edge_index[1].astype(jnp.int32).reshape(NW, EPW)
  pad_dst = N + (jnp.arange(EPAD, dtype=jnp.int32) % 224)
  pad_src = jnp.arange(EPAD, dtype=jnp.int32) * 41 % N
  src = jnp.concatenate(
      [src, jnp.broadcast_to(pad_src, (NW, EPAD))], axis=1
  ).reshape(NW, NWIN, WCH, CS)
  dst = jnp.concatenate(
      [dst, jnp.broadcast_to(pad_dst, (NW, EPAD))], axis=1
  ).reshape(NW, NWIN, WCH, CS)
  Wl = Wl_stack[layer]
  Wr = Wr_stack[layer]
  b = b_stack[layer]

  aggr1, degp = _agg_with_deg(x, src, dst)
  h = _mm(aggr1, degp, x, Wl, Wr, b.reshape(1, D), relu=True)
  (aggr2,) = _agg_no_deg(h, src, dst)
  out = _mm(aggr2, degp, h, W2_l, W2_r, b2.reshape(1, D), relu=False)
  return out


# speedup vs baseline: 3.0536x; 1.0513x over previous
import functools

import jax
import jax.numpy as jnp
from jax import lax
from jax.experimental import pallas as pl
from jax.experimental.pallas import tpu as pltpu
from jax.experimental.pallas import tpu_sc as plsc

N = 10000
NP = 10240
D = 128
E = 320000
NC = 2
NS = 16
NW = NC * NS
EPW = E // NW
CS = 128
WCH = 8
NWIN = 10
NCH = NWIN * WCH
EPAD = 240
RPS = NP // NS
BN = 1024

_mesh = plsc.VectorSubcoreMesh(core_axis_name="c", subcore_axis_name="s")


def _make_agg(compute_deg):
  out_type = [jax.ShapeDtypeStruct((NC, NP, D), jnp.float32)]
  scratch = [
      pltpu.VMEM((2, WCH, CS), jnp.int32),
      pltpu.VMEM((2, WCH, CS), jnp.int32),
      pltpu.VMEM((2, CS, D), jnp.float32),
      pltpu.VMEM_SHARED((NP, D), jnp.float32),
      pltpu.SemaphoreType.DMA,
      pltpu.SemaphoreType.DMA,
      pltpu.SemaphoreType.DMA,
  ]
  if compute_deg:
    out_type.append(jax.ShapeDtypeStruct((NW, NP), jnp.float32))
    scratch.insert(3, pltpu.VMEM((NP,), jnp.float32))

  @functools.partial(pl.kernel, mesh=_mesh, out_type=out_type,
                     scratch_types=scratch)
  def agg(*refs):
    if compute_deg:
      (x_hbm, src_hbm, dst_hbm, aggr_out, deg_out,
       src_v, dst_v, rows_v, deg_v, acc_sh, sem_g, sem_s, sem_i) = refs
    else:
      (x_hbm, src_hbm, dst_hbm, aggr_out,
       src_v, dst_v, rows_v, acc_sh, sem_g, sem_s, sem_i) = refs

    c = lax.axis_index("c")
    s = lax.axis_index("s")
    w = c * NS + s

    z16 = jnp.zeros((16,), jnp.float32)

    def zrow(i, carry):
      for l in range(D // 16):
        rows_v[0, i, pl.ds(l * 16, 16)] = z16
      return carry
    lax.fori_loop(0, CS, zrow, 0)
    for t in range(RPS // CS):
      pltpu.sync_copy(rows_v.at[0], acc_sh.at[pl.ds(s * RPS + t * CS, CS)])
    if compute_deg:
      def zdeg(i, carry):
        deg_v[pl.ds(i * 16, 16)] = z16
        return carry
      lax.fori_loop(0, NP // 16, zdeg, 0)
    plsc.subcore_barrier()

    onehot = jnp.where(lax.iota(jnp.int32, 16) == 0, 1.0, 0.0)

    def hist(b, j):
      if compute_deg:
        for l in range(CS // 16):
          dvec = dst_v[b, j, pl.ds(l * 16, 16)]
          for q in range(16):
            d = dvec[q]
            deg_v[pl.ds(d, 16)] = deg_v[pl.ds(d, 16)] + onehot

    def drain(sem):
      pltpu.make_async_copy(x_hbm.at[src_v.at[0].at[0]], rows_v.at[0],
                            sem).wait()

    def drain_idx():
      pltpu.make_async_copy(src_hbm.at[w, 0], src_v.at[0], sem_i).wait()

    pltpu.sync_copy(src_hbm.at[w, 0], src_v.at[0])
    pltpu.sync_copy(dst_hbm.at[w, 0], dst_v.at[0])
    pltpu.async_copy(x_hbm.at[src_v.at[0].at[0]], rows_v.at[0], sem_g).wait()

    def chunk(g, carry):
      t = g // WCH
      j = lax.rem(g, WCH)
      b = lax.rem(t, 2)
      p = lax.rem(g, 2)
      tn = (g + 1) // WCH
      jn = lax.rem(g + 1, WCH)
      bn = lax.rem(tn, 2)

      @pl.when(jnp.logical_and(j == 0, t + 1 < NWIN))
      def _():
        pltpu.async_copy(src_hbm.at[w, t + 1], src_v.at[1 - b], sem_i)
        pltpu.async_copy(dst_hbm.at[w, t + 1], dst_v.at[1 - b], sem_i)

      @pl.when(jnp.logical_and(g + 1 < NCH, jn == 0))
      def _():
        drain_idx()
        drain_idx()

      @pl.when(g + 1 < NCH)
      def _():
        pltpu.async_copy(x_hbm.at[src_v.at[bn].at[jn]], rows_v.at[1 - p],
                         sem_g)
      pltpu.async_copy(rows_v.at[p], acc_sh.at[dst_v.at[b].at[j]], sem_s,
                       add=True)
      hist(b, j)
      drain(sem_s)

      @pl.when(g + 1 < NCH)
      def _():
        drain(sem_g)
      return carry
    lax.fori_loop(0, NCH, chunk, 0)

    plsc.subcore_barrier()
    pltpu.sync_copy(acc_sh.at[pl.ds(s * RPS, RPS)],
                    aggr_out.at[c, pl.ds(s * RPS, RPS)])
    if compute_deg:
      pltpu.sync_copy(deg_v, deg_out.at[w])

  return agg


_agg_with_deg = _make_agg(True)
_agg_no_deg = _make_agg(False)


def _mm_body(relu, aggrp_ref, deg_ref, x_ref, wl_ref, wr_ref, b_ref, out_ref):
  a = aggrp_ref[0] + aggrp_ref[1]
  deg = jnp.sum(deg_ref[...], axis=0)
  inv = 1.0 / jnp.maximum(deg, 1.0)
  m = a * inv[:, None]
  acc = jnp.dot(m, wl_ref[...], preferred_element_type=jnp.float32)
  acc = acc + jnp.dot(x_ref[...], wr_ref[...],
                      preferred_element_type=jnp.float32)
  acc = acc + b_ref[...]
  if relu:
    acc = jnp.maximum(acc, 0.0)
  out_ref[...] = acc


def _mm(aggrp, degp, x, wl, wr, b, relu):
  grid = (pl.cdiv(N, BN),)
  return pl.pallas_call(
      functools.partial(_mm_body, relu),
      grid=grid,
      in_specs=[
          pl.BlockSpec((NC, BN, D), lambda i: (0, i, 0)),
          pl.BlockSpec((NW, BN), lambda i: (0, i)),
          pl.BlockSpec((BN, D), lambda i: (i, 0)),
          pl.BlockSpec((D, D), lambda i: (0, 0)),
          pl.BlockSpec((D, D), lambda i: (0, 0)),
          pl.BlockSpec((1, D), lambda i: (0, 0)),
      ],
      out_specs=pl.BlockSpec((BN, D), lambda i: (i, 0)),
      out_shape=jax.ShapeDtypeStruct((N, D), jnp.float32),
  )(aggrp, degp, x, wl, wr, b)


def kernel(x, edge_index, layer, Wl_stack, Wr_stack, b_stack, W2_l, W2_r, b2):
  src = edge_index[0].astype(jnp.int32).reshape(NW, EPW)
  dst = edge_index[1].astype(jnp.int32).reshape(NW, EPW)
  pad_dst = N + (jnp.arange(EPAD, dtype=jnp.int32) % 224)
  pad_src = jnp.arange(EPAD, dtype=jnp.int32) * 41 % N
  src = jnp.concatenate(
      [src, jnp.broadcast_to(pad_src, (NW, EPAD))], axis=1
  ).reshape(NW, NWIN, WCH, CS)
  dst = jnp.concatenate(
      [dst, jnp.broadcast_to(pad_dst, (NW, EPAD))], axis=1
  ).reshape(NW, NWIN, WCH, CS)
  Wl = Wl_stack[layer]
  Wr = Wr_stack[layer]
  b = b_stack[layer]

  aggr1, degp = _agg_with_deg(x, src, dst)
  h = _mm(aggr1, degp, x, Wl, Wr, b.reshape(1, D), relu=True)
  (aggr2,) = _agg_no_deg(h, src, dst)
  out = _mm(aggr2, degp, h, W2_l, W2_r, b2.reshape(1, D), relu=False)
  return out
